# Initial kernel scaffold; baseline (speedup 1.0000x reference)
import jax
import jax.numpy as jnp
from jax.experimental import pallas as pl


def kernel(pos, edge_index, batch, W1, b1, gamma, beta, W2, b2):
    raise NotImplementedError("write your pallas kernel here")


# speedup vs baseline: 3.1641x; 3.1641x over previous
import jax
import jax.numpy as jnp
from jax import lax
from jax.experimental import pallas as pl
from jax.experimental.pallas import tpu as pltpu
from jax.experimental.pallas import tpu_sc as plsc

N = 50000
E = 800000
H2 = 64
ROW = 128
NROWS = 6400
EPAD = NROWS * ROW
NC, NS = 2, 16
NW = NC * NS
HALF = N // NC
ACC_ROWS = 25128
ZSTRIDE = 1576
BIGIDX = 2_000_000

RPW = NROWS // NW
KBA = 8
PW = 16
PSTRIPE = 3128

HROWS = NROWS // 2
CROWS = EPAD // 2
CRPT = HROWS // NS
CB = 112
ZCH = 15
DCH = 14
TSTRIDE = 1568

_SC_PARAMS = pltpu.CompilerParams(use_tc_tiling_on_sc=False,
                                  needs_layout_passes=False)


def _rsqrt_sc(d2):
    bits = plsc.bitcast(d2, jnp.int32)
    y = plsc.bitcast(0x5F3759DF - (bits >> 1), jnp.float32)
    for _ in range(3):
        y = y * (1.5 - (0.5 * d2 * y) * y)
    return y


def _gather_body(pos_hbm, src_hbm, dst_hbm, posU_hbm,
                 sidx, didx, sbuf, dbuf, planes, pos_sh, sem):
    c = lax.axis_index("c")
    s = lax.axis_index("s")
    wid = s * NC + c
    iota = lax.iota(jnp.int32, 16)

    soff = jnp.minimum(s * PSTRIPE, N - PSTRIPE)
    pltpu.sync_copy(pos_hbm.at[pl.ds(soff, PSTRIPE)],
                    pos_sh.at[pl.ds(soff, PSTRIPE)])
    plsc.subcore_barrier()

    def block(b, carry):
        r0 = wid * RPW + b * KBA
        pltpu.sync_copy(src_hbm.at[pl.ds(r0, KBA)], sidx)
        pltpu.sync_copy(dst_hbm.at[pl.ds(r0, KBA)], didx)
        descs = []
        for j in range(KBA):
            descs.append(pltpu.async_copy(
                pos_sh.at[sidx.at[j]], sbuf.at[pl.ds(j * ROW, ROW)], sem))
            descs.append(pltpu.async_copy(
                pos_sh.at[didx.at[j]], dbuf.at[pl.ds(j * ROW, ROW)], sem))
        for d in descs:
            d.wait()

        def grp(g, carry2):
            ridx = g * 16 + iota
            comp = []
            for cc in range(3):
                cidx = jnp.full((16,), cc, jnp.int32)
                xs = plsc.load_gather(sbuf, [ridx, cidx])
                xd = plsc.load_gather(dbuf, [ridx, cidx])
                comp.append(xd - xs)
            d2 = comp[0] * comp[0] + comp[1] * comp[1] + comp[2] * comp[2]
            r = _rsqrt_sc(d2)
            f = 1.0 / (d2 * r + 1e-6)
            for cc in range(3):
                planes[cc, pl.ds(g * 16, 16)] = comp[cc] * f
            return carry2
        lax.fori_loop(0, KBA * ROW // 16, grp, 0)
        for cc in range(3):
            pltpu.sync_copy(planes.at[cc],
                            posU_hbm.at[cc, pl.ds(r0 * ROW, KBA * ROW)])
        return carry

    lax.fori_loop(0, RPW // KBA, block, 0)


@jax.jit
def _gather_call(pos16, src2d, dst2d):
    f = pl.kernel(
        _gather_body,
        out_type=jax.ShapeDtypeStruct((8, EPAD), jnp.float32),
        mesh=plsc.VectorSubcoreMesh(core_axis_name="c", subcore_axis_name="s"),
        scratch_types=[
            pltpu.VMEM((KBA, ROW), jnp.int32),
            pltpu.VMEM((KBA, ROW), jnp.int32),
            pltpu.VMEM((KBA * ROW, PW), jnp.float32),
            pltpu.VMEM((KBA * ROW, PW), jnp.float32),
            pltpu.VMEM((3, KBA * ROW), jnp.float32),
            pltpu.VMEM_SHARED((N, PW), jnp.float32),
            pltpu.SemaphoreType.DMA,
        ],
        compiler_params=_SC_PARAMS,
    )
    return f(pos16, src2d, dst2d)


BE = 1024


def _mlp_body(w1_ref, b1_ref, gam_ref, bet_ref, w2_ref, b2_ref,
              ul_ref, ur_ref, o_ref):
    w1 = w1_ref[...]
    w2 = w2_ref[...]
    b1 = b1_ref[...]
    gam = gam_ref[...]
    bet = bet_ref[...]
    b2 = b2_ref[...]

    def half(u_ref):
        u = u_ref[0:3, :]
        h = lax.dot_general(u, w1, (((0,), (0,)), ((), ())),
                            preferred_element_type=jnp.float32) + b1
        mu = jnp.mean(h, axis=1, keepdims=True)
        cen = h - mu
        var = jnp.mean(cen * cen, axis=1, keepdims=True)
        hn = cen * lax.rsqrt(var + 1e-5) * gam + bet
        g = 0.5 * hn * (1.0 + lax.erf(hn * 0.7071067811865476))
        return jnp.dot(g, w2, preferred_element_type=jnp.float32) + b2

    o_ref[...] = jnp.concatenate([half(ul_ref), half(ur_ref)], axis=1)


@jax.jit
def _mlp_call(W1, b1, gamma, beta, W2, b2, posU):
    grid = (CROWS // BE,)
    full = lambda i: (0, 0)
    return pl.pallas_call(
        _mlp_body,
        grid=grid,
        in_specs=[
            pl.BlockSpec((3, H2), full),
            pl.BlockSpec((1, H2), full),
            pl.BlockSpec((1, H2), full),
            pl.BlockSpec((1, H2), full),
            pl.BlockSpec((H2, H2), full),
            pl.BlockSpec((1, H2), full),
            pl.BlockSpec((8, BE), lambda i: (0, i)),
            pl.BlockSpec((8, BE), lambda i: (0, i + CROWS // BE)),
        ],
        out_specs=pl.BlockSpec((BE, 2 * H2), lambda i: (i, 0)),
        out_shape=jax.ShapeDtypeStruct((CROWS, 2 * H2), jnp.float32),
    )(W1, b1.reshape(1, H2), gamma.reshape(1, H2), beta.reshape(1, H2),
      W2, b2.reshape(1, H2), posU, posU)


def _scatter_body(codes_hbm, dstC_hbm, out_hbm,
                  idxL, idxR, updL, updR, ones_v, obuf, degb, acc, deg):
    c = lax.axis_index("c")
    t = lax.axis_index("s")
    base = c * HALF
    iota = lax.iota(jnp.int32, 16)
    zero16 = jnp.zeros((16,), jnp.float32)

    one16 = jnp.full((16,), 1.0, jnp.float32)

    def initrow(r, carry):
        for g in range(4):
            obuf[r, pl.ds(g * 16, 16)] = zero16
        return carry
    lax.fori_loop(0, CB, initrow, 0)
    for k in range(CB // 16):
        degb[pl.ds(k * 16, 16)] = zero16
    for k in range(ROW // 16):
        ones_v[pl.ds(k * 16, 16)] = one16

    def zchunk(i, carry):
        cs = jnp.minimum(t * ZSTRIDE + i * CB, ACC_ROWS - CB)
        pltpu.sync_copy(obuf, acc.at[pl.ds(cs, CB)])
        pltpu.sync_copy(degb, deg.at[pl.ds(cs, CB)])
        return carry
    lax.fori_loop(0, ZCH, zchunk, 0)
    plsc.subcore_barrier()

    def remap(idx):
        for i in range(8):
            v = idx[pl.ds(i * 16, 16)]
            inr = jnp.logical_and(v >= base, v < base + HALF)
            dumped = HALF + i * 16 + iota
            idx[pl.ds(i * 16, 16)] = jnp.where(inr, v - base, dumped)

    def sblock(b, carry):
        j = t * CRPT + b
        pltpu.sync_copy(dstC_hbm.at[j], idxL)
        pltpu.sync_copy(dstC_hbm.at[j + HROWS], idxR)
        remap(idxL)
        remap(idxR)
        pltpu.sync_copy(codes_hbm.at[pl.ds(j * ROW, ROW), pl.ds(0, H2)], updL)
        pltpu.sync_copy(codes_hbm.at[pl.ds(j * ROW, ROW), pl.ds(H2, H2)], updR)
        pltpu.sync_copy(updL, acc.at[idxL], add=True)
        pltpu.sync_copy(ones_v, deg.at[idxL], add=True)
        pltpu.sync_copy(updR, acc.at[idxR], add=True)
        pltpu.sync_copy(ones_v, deg.at[idxR], add=True)
        return carry
    lax.fori_loop(0, CRPT, sblock, 0)
    plsc.subcore_barrier()

    def dchunk(i, carry):
        cs = jnp.minimum(t * TSTRIDE + i * CB, HALF - CB)
        pltpu.sync_copy(acc.at[pl.ds(cs, CB)], obuf)
        pltpu.sync_copy(deg.at[pl.ds(cs, CB)], degb)

        def drow(r, carry2):
            dvec = plsc.load_gather(degb, [jnp.full((16,), r, jnp.int32)])
            rec = 1.0 / jnp.maximum(dvec, 1.0)
            for g in range(4):
                obuf[r, pl.ds(g * 16, 16)] = obuf[r, pl.ds(g * 16, 16)] * rec
            return carry2
        lax.fori_loop(0, CB, drow, 0)
        pltpu.sync_copy(obuf, out_hbm.at[pl.ds(base + cs, CB)])
        return carry
    lax.fori_loop(0, DCH, dchunk, 0)


@jax.jit
def _scatter_call(codes, dstC):
    f = pl.kernel(
        _scatter_body,
        out_type=jax.ShapeDtypeStruct((N, H2), jnp.float32),
        mesh=plsc.VectorSubcoreMesh(core_axis_name="c", subcore_axis_name="s"),
        scratch_types=[
            pltpu.VMEM((ROW,), jnp.int32),
            pltpu.VMEM((ROW,), jnp.int32),
            pltpu.VMEM((ROW, H2), jnp.float32),
            pltpu.VMEM((ROW, H2), jnp.float32),
            pltpu.VMEM((ROW,), jnp.float32),
            pltpu.VMEM((CB, H2), jnp.float32),
            pltpu.VMEM((CB,), jnp.float32),
            pltpu.VMEM_SHARED((ACC_ROWS, H2), jnp.float32),
            pltpu.VMEM_SHARED((ACC_ROWS,), jnp.float32),
        ],
        compiler_params=_SC_PARAMS,
    )
    return f(codes, dstC)


def _jnp_mlp(u, W1, b1, gamma, beta, W2, b2):
    h = u @ W1 + b1
    mu = jnp.mean(h, axis=-1, keepdims=True)
    var = jnp.var(h, axis=-1, keepdims=True)
    h = (h - mu) / jnp.sqrt(var + 1e-5) * gamma + beta
    h = jax.nn.gelu(h, approximate=False)
    return h @ W2 + b2


DEBUG_STAGE = 0


def kernel(pos, edge_index, batch, W1, b1, gamma, beta, W2, b2):
    src = edge_index[0]
    dst = edge_index[1]
    pos16 = jnp.pad(pos, ((0, 0), (0, PW - 3)))
    pad = EPAD - E
    src2d = jnp.pad(src, (0, pad)).reshape(NROWS, ROW)
    dstA = jnp.pad(dst, (0, pad)).reshape(NROWS, ROW)
    dstC = jnp.pad(dst, (0, pad), constant_values=BIGIDX).reshape(NROWS, ROW)
    if DEBUG_STAGE == 1:
        posU = _gather_call(pos16, src2d, dstA)
        u = posU[:3, :E].T
        code = _jnp_mlp(u, W1, b1, gamma, beta, W2, b2)
        pe = jnp.zeros((N, H2), jnp.float32).at[dst].add(code)
        degs = jnp.zeros((N,), jnp.float32).at[dst].add(1.0)
        return pe / jnp.maximum(degs, 1.0)[:, None]
    if DEBUG_STAGE == 2:
        rel = pos[dst] - pos[src]
        dist = jnp.linalg.norm(rel, axis=1, keepdims=True)
        u = rel / (dist + 1e-6)
        uP = jnp.pad(u, ((0, EPAD - E), (0, 0))).T
        uP = jnp.pad(uP, ((0, 5), (0, 0)))
        codes = _mlp_call(W1, b1, gamma, beta, W2, b2, uP)
        code = jnp.concatenate(
            [codes[:, :H2], codes[:, H2:]], axis=0)[:E]
        pe = jnp.zeros((N, H2), jnp.float32).at[dst].add(code)
        degs = jnp.zeros((N,), jnp.float32).at[dst].add(1.0)
        return pe / jnp.maximum(degs, 1.0)[:, None]
    if DEBUG_STAGE == 3:
        rel = pos[dst] - pos[src]
        dist = jnp.linalg.norm(rel, axis=1, keepdims=True)
        u = rel / (dist + 1e-6)
        code = _jnp_mlp(u, W1, b1, gamma, beta, W2, b2)
        codesP = jnp.pad(code, ((0, EPAD - E), (0, 0)))
        codes = jnp.concatenate(
            [codesP[:EPAD // 2], codesP[EPAD // 2:]], axis=1)
        return _scatter_call(codes, dstC)
    posU = _gather_call(pos16, src2d, dstA)
    codes = _mlp_call(W1, b1, gamma, beta, W2, b2, posU)
    return _scatter_call(codes, dstC)


# speedup vs baseline: 5.6733x; 1.7930x over previous
import jax
import jax.numpy as jnp
from jax import lax
from jax.experimental import pallas as pl
from jax.experimental.pallas import tpu as pltpu
from jax.experimental.pallas import tpu_sc as plsc

N = 50000
E = 800000
H2 = 64
ROW = 128
NROWS = 6400
EPAD = NROWS * ROW
NC, NS = 2, 16
NW = NC * NS
HALF = N // NC
ACC_ROWS = 25128
ZSTRIDE = 1576
BIGIDX = 2_000_000

RPW = NROWS // NW
KBA = 8
PW = 16
PSTRIPE = 3128

HROWS = NROWS // 2
CROWS = EPAD // 2
CRPT = HROWS // NS
CB = 112
ZCH = 15
DCH = 14
TSTRIDE = 1568

_SC_PARAMS = pltpu.CompilerParams(use_tc_tiling_on_sc=False,
                                  needs_layout_passes=False)


def _rsqrt_sc(d2):
    bits = plsc.bitcast(d2, jnp.int32)
    y = plsc.bitcast(0x5F3759DF - (bits >> 1), jnp.float32)
    for _ in range(3):
        y = y * (1.5 - (0.5 * d2 * y) * y)
    return y


def _gather_body(pos_hbm, src_hbm, dst_hbm, posU_hbm,
                 sidx, didx, sbuf, dbuf, planes, pos_sh, sem):
    c = lax.axis_index("c")
    s = lax.axis_index("s")
    wid = s * NC + c
    iota = lax.iota(jnp.int32, 16)

    soff = jnp.minimum(s * PSTRIPE, N - PSTRIPE)
    pltpu.sync_copy(pos_hbm.at[pl.ds(soff, PSTRIPE)],
                    pos_sh.at[pl.ds(soff, PSTRIPE)])
    plsc.subcore_barrier()

    def block(b, carry):
        r0 = wid * RPW + b * KBA
        pltpu.sync_copy(src_hbm.at[pl.ds(r0, KBA)], sidx)
        pltpu.sync_copy(dst_hbm.at[pl.ds(r0, KBA)], didx)
        descs = []
        for j in range(KBA):
            descs.append(pltpu.async_copy(
                pos_sh.at[sidx.at[j]], sbuf.at[pl.ds(j * ROW, ROW)], sem))
            descs.append(pltpu.async_copy(
                pos_sh.at[didx.at[j]], dbuf.at[pl.ds(j * ROW, ROW)], sem))
        for d in descs:
            d.wait()

        def grp(g, carry2):
            ridx = g * 16 + iota
            grow = 3 * (g // 8)
            lo = (g % 8) * 16
            comp = []
            for cc in range(3):
                cidx = jnp.full((16,), cc, jnp.int32)
                xs = plsc.load_gather(sbuf, [ridx, cidx])
                xd = plsc.load_gather(dbuf, [ridx, cidx])
                comp.append(xd - xs)
            d2 = comp[0] * comp[0] + comp[1] * comp[1] + comp[2] * comp[2]
            r = _rsqrt_sc(d2)
            f = 1.0 / (d2 * r + 1e-6)
            for cc in range(3):
                planes[grow + cc, pl.ds(lo, 16)] = comp[cc] * f
            return carry2
        lax.fori_loop(0, KBA * ROW // 16, grp, 0)
        pltpu.sync_copy(planes, posU_hbm.at[pl.ds(3 * r0, 3 * KBA)])
        return carry

    lax.fori_loop(0, RPW // KBA, block, 0)


@jax.jit
def _gather_call(pos16, src2d, dst2d):
    f = pl.kernel(
        _gather_body,
        out_type=jax.ShapeDtypeStruct((3 * NROWS, ROW), jnp.float32),
        mesh=plsc.VectorSubcoreMesh(core_axis_name="c", subcore_axis_name="s"),
        scratch_types=[
            pltpu.VMEM((KBA, ROW), jnp.int32),
            pltpu.VMEM((KBA, ROW), jnp.int32),
            pltpu.VMEM((KBA * ROW, PW), jnp.float32),
            pltpu.VMEM((KBA * ROW, PW), jnp.float32),
            pltpu.VMEM((3 * KBA, ROW), jnp.float32),
            pltpu.VMEM_SHARED((N, PW), jnp.float32),
            pltpu.SemaphoreType.DMA,
        ],
        compiler_params=_SC_PARAMS,
    )
    return f(pos16, src2d, dst2d)


BE = 1024


def _mlp_body(w1_ref, b1_ref, gam_ref, bet_ref, w2_ref, b2_ref,
              ul_ref, ur_ref, o_ref):
    w1 = w1_ref[...]
    w2 = w2_ref[...]
    b1 = b1_ref[...]
    gam = gam_ref[...]
    bet = bet_ref[...]
    b2 = b2_ref[...]

    def half(u_ref):
        hs = []
        for g in range(BE // ROW):
            ug = u_ref[3 * g:3 * g + 3, :]
            hs.append(lax.dot_general(ug, w1, (((0,), (0,)), ((), ())),
                                      preferred_element_type=jnp.float32))
        h = jnp.concatenate(hs, axis=0) + b1
        mu = jnp.mean(h, axis=1, keepdims=True)
        cen = h - mu
        var = jnp.mean(cen * cen, axis=1, keepdims=True)
        hn = cen * lax.rsqrt(var + 1e-5) * gam + bet
        g = 0.5 * hn * (1.0 + lax.erf(hn * 0.7071067811865476))
        return jnp.dot(g, w2, preferred_element_type=jnp.float32) + b2

    o_ref[...] = jnp.concatenate([half(ul_ref), half(ur_ref)], axis=1)


@jax.jit
def _mlp_call(W1, b1, gamma, beta, W2, b2, posU):
    grid = (CROWS // BE,)
    full = lambda i: (0, 0)
    return pl.pallas_call(
        _mlp_body,
        grid=grid,
        in_specs=[
            pl.BlockSpec((3, H2), full),
            pl.BlockSpec((1, H2), full),
            pl.BlockSpec((1, H2), full),
            pl.BlockSpec((1, H2), full),
            pl.BlockSpec((H2, H2), full),
            pl.BlockSpec((1, H2), full),
            pl.BlockSpec((3 * BE // ROW, ROW), lambda i: (i, 0)),
            pl.BlockSpec((3 * BE // ROW, ROW),
                         lambda i: (i + CROWS // BE, 0)),
        ],
        out_specs=pl.BlockSpec((BE, 2 * H2), lambda i: (i, 0)),
        out_shape=jax.ShapeDtypeStruct((CROWS, 2 * H2), jnp.float32),
    )(W1, b1.reshape(1, H2), gamma.reshape(1, H2), beta.reshape(1, H2),
      W2, b2.reshape(1, H2), posU, posU)


def _scatter_body(codes_hbm, dstC_hbm, out_hbm,
                  idxL, idxR, updL, updR, ones_v, obuf, degb, acc, deg):
    c = lax.axis_index("c")
    t = lax.axis_index("s")
    base = c * HALF
    iota = lax.iota(jnp.int32, 16)
    zero16 = jnp.zeros((16,), jnp.float32)

    one16 = jnp.full((16,), 1.0, jnp.float32)

    def initrow(r, carry):
        for g in range(4):
            obuf[r, pl.ds(g * 16, 16)] = zero16
        return carry
    lax.fori_loop(0, CB, initrow, 0)
    for k in range(CB // 16):
        degb[pl.ds(k * 16, 16)] = zero16
    for k in range(ROW // 16):
        ones_v[pl.ds(k * 16, 16)] = one16

    def zchunk(i, carry):
        cs = jnp.minimum(t * ZSTRIDE + i * CB, ACC_ROWS - CB)
        pltpu.sync_copy(obuf, acc.at[pl.ds(cs, CB)])
        pltpu.sync_copy(degb, deg.at[pl.ds(cs, CB)])
        return carry
    lax.fori_loop(0, ZCH, zchunk, 0)
    plsc.subcore_barrier()

    def remap(idx):
        for i in range(8):
            v = idx[pl.ds(i * 16, 16)]
            inr = jnp.logical_and(v >= base, v < base + HALF)
            dumped = HALF + i * 16 + iota
            idx[pl.ds(i * 16, 16)] = jnp.where(inr, v - base, dumped)

    def sblock(b, carry):
        j = t * CRPT + b
        pltpu.sync_copy(dstC_hbm.at[j], idxL)
        pltpu.sync_copy(dstC_hbm.at[j + HROWS], idxR)
        remap(idxL)
        remap(idxR)
        pltpu.sync_copy(codes_hbm.at[pl.ds(j * ROW, ROW), pl.ds(0, H2)], updL)
        pltpu.sync_copy(codes_hbm.at[pl.ds(j * ROW, ROW), pl.ds(H2, H2)], updR)
        pltpu.sync_copy(updL, acc.at[idxL], add=True)
        pltpu.sync_copy(ones_v, deg.at[idxL], add=True)
        pltpu.sync_copy(updR, acc.at[idxR], add=True)
        pltpu.sync_copy(ones_v, deg.at[idxR], add=True)
        return carry
    lax.fori_loop(0, CRPT, sblock, 0)
    plsc.subcore_barrier()

    def dchunk(i, carry):
        cs = jnp.minimum(t * TSTRIDE + i * CB, HALF - CB)
        pltpu.sync_copy(acc.at[pl.ds(cs, CB)], obuf)
        pltpu.sync_copy(deg.at[pl.ds(cs, CB)], degb)

        def drow(r, carry2):
            dvec = plsc.load_gather(degb, [jnp.full((16,), r, jnp.int32)])
            rec = 1.0 / jnp.maximum(dvec, 1.0)
            for g in range(4):
                obuf[r, pl.ds(g * 16, 16)] = obuf[r, pl.ds(g * 16, 16)] * rec
            return carry2
        lax.fori_loop(0, CB, drow, 0)
        pltpu.sync_copy(obuf, out_hbm.at[pl.ds(base + cs, CB)])
        return carry
    lax.fori_loop(0, DCH, dchunk, 0)


@jax.jit
def _scatter_call(codes, dstC):
    f = pl.kernel(
        _scatter_body,
        out_type=jax.ShapeDtypeStruct((N, H2), jnp.float32),
        mesh=plsc.VectorSubcoreMesh(core_axis_name="c", subcore_axis_name="s"),
        scratch_types=[
            pltpu.VMEM((ROW,), jnp.int32),
            pltpu.VMEM((ROW,), jnp.int32),
            pltpu.VMEM((ROW, H2), jnp.float32),
            pltpu.VMEM((ROW, H2), jnp.float32),
            pltpu.VMEM((ROW,), jnp.float32),
            pltpu.VMEM((CB, H2), jnp.float32),
            pltpu.VMEM((CB,), jnp.float32),
            pltpu.VMEM_SHARED((ACC_ROWS, H2), jnp.float32),
            pltpu.VMEM_SHARED((ACC_ROWS,), jnp.float32),
        ],
        compiler_params=_SC_PARAMS,
    )
    return f(codes, dstC)


def _jnp_mlp(u, W1, b1, gamma, beta, W2, b2):
    h = u @ W1 + b1
    mu = jnp.mean(h, axis=-1, keepdims=True)
    var = jnp.var(h, axis=-1, keepdims=True)
    h = (h - mu) / jnp.sqrt(var + 1e-5) * gamma + beta
    h = jax.nn.gelu(h, approximate=False)
    return h @ W2 + b2


DEBUG_STAGE = 0


def kernel(pos, edge_index, batch, W1, b1, gamma, beta, W2, b2):
    src = edge_index[0]
    dst = edge_index[1]
    pos16 = jnp.pad(pos, ((0, 0), (0, PW - 3)))
    pad = EPAD - E
    src2d = jnp.pad(src, (0, pad)).reshape(NROWS, ROW)
    dstA = jnp.pad(dst, (0, pad)).reshape(NROWS, ROW)
    dstC = jnp.pad(dst, (0, pad), constant_values=BIGIDX).reshape(NROWS, ROW)
    if DEBUG_STAGE == 1:
        posU = _gather_call(pos16, src2d, dstA)
        u = posU[:3, :E].T
        code = _jnp_mlp(u, W1, b1, gamma, beta, W2, b2)
        pe = jnp.zeros((N, H2), jnp.float32).at[dst].add(code)
        degs = jnp.zeros((N,), jnp.float32).at[dst].add(1.0)
        return pe / jnp.maximum(degs, 1.0)[:, None]
    if DEBUG_STAGE == 2:
        rel = pos[dst] - pos[src]
        dist = jnp.linalg.norm(rel, axis=1, keepdims=True)
        u = rel / (dist + 1e-6)
        uP = jnp.pad(u, ((0, EPAD - E), (0, 0)))
        uP = uP.reshape(NROWS, ROW, 3).transpose(0, 2, 1).reshape(
            3 * NROWS, ROW)
        codes = _mlp_call(W1, b1, gamma, beta, W2, b2, uP)
        code = jnp.concatenate(
            [codes[:, :H2], codes[:, H2:]], axis=0)[:E]
        pe = jnp.zeros((N, H2), jnp.float32).at[dst].add(code)
        degs = jnp.zeros((N,), jnp.float32).at[dst].add(1.0)
        return pe / jnp.maximum(degs, 1.0)[:, None]
    if DEBUG_STAGE == 3:
        rel = pos[dst] - pos[src]
        dist = jnp.linalg.norm(rel, axis=1, keepdims=True)
        u = rel / (dist + 1e-6)
        code = _jnp_mlp(u, W1, b1, gamma, beta, W2, b2)
        codesP = jnp.pad(code, ((0, EPAD - E), (0, 0)))
        codes = jnp.concatenate(
            [codesP[:EPAD // 2], codesP[EPAD // 2:]], axis=1)
        return _scatter_call(codes, dstC)
    posU = _gather_call(pos16, src2d, dstA)
    codes = _mlp_call(W1, b1, gamma, beta, W2, b2, posU)
    return _scatter_call(codes, dstC)


# speedup vs baseline: 7.7869x; 1.3726x over previous
import jax
import jax.numpy as jnp
from jax import lax
from jax.experimental import pallas as pl
from jax.experimental.pallas import tpu as pltpu
from jax.experimental.pallas import tpu_sc as plsc

N = 50000
E = 800000
H2 = 64
ROW = 128
NROWS = 6400
EPAD = NROWS * ROW
RROWS = E // ROW
NC, NS = 2, 16
NW = NC * NS
HALF = N // NC
ACC_ROWS = 25128
BIGROW = 25000
CROWS = EPAD // 2

RPW = 196
KBA = 7
PW = 16
PSTRIPE = 3128

HROWS = NROWS // 2
CRPT = HROWS // NS
RREAL = 3050
CB = 48
ZCH = 33
DCH = 33
TSTRIDE = 1568
ZSTRIDE = 1576

_SC_PARAMS = pltpu.CompilerParams(use_tc_tiling_on_sc=False,
                                  needs_layout_passes=False)


def _rsqrt_sc(d2):
    bits = plsc.bitcast(d2, jnp.int32)
    y = plsc.bitcast(0x5F3759DF - (bits >> 1), jnp.float32)
    for _ in range(3):
        y = y * (1.5 - (0.5 * d2 * y) * y)
    return y


def _gather_body(pos_hbm, src_hbm, dst_hbm, posU_hbm,
                 sidx, didx, sbuf, dbuf, planes, pos_sh, sem):
    c = lax.axis_index("c")
    s = lax.axis_index("s")
    wid = s * NC + c
    iota = lax.iota(jnp.int32, 16)

    soff = jnp.minimum(s * PSTRIPE, N - PSTRIPE)
    pltpu.sync_copy(pos_hbm.at[pl.ds(soff, PSTRIPE)],
                    pos_sh.at[pl.ds(soff, PSTRIPE)])
    plsc.subcore_barrier()

    start = jnp.minimum(wid * RPW, RROWS - RPW)

    def block(b, carry):
        r0 = start + b * KBA
        pltpu.sync_copy(src_hbm.at[pl.ds(r0 * ROW, KBA * ROW)], sidx)
        pltpu.sync_copy(dst_hbm.at[pl.ds(r0 * ROW, KBA * ROW)], didx)
        descs = []
        for j in range(KBA):
            descs.append(pltpu.async_copy(
                pos_sh.at[sidx.at[pl.ds(j * ROW, ROW)]],
                sbuf.at[pl.ds(j * ROW, ROW)], sem))
            descs.append(pltpu.async_copy(
                pos_sh.at[didx.at[pl.ds(j * ROW, ROW)]],
                dbuf.at[pl.ds(j * ROW, ROW)], sem))
        for d in descs:
            d.wait()

        def grp(g, carry2):
            ridx = g * 16 + iota
            grow = 3 * (g // 8)
            lo = (g % 8) * 16
            comp = []
            for cc in range(3):
                cidx = jnp.full((16,), cc, jnp.int32)
                xs = plsc.load_gather(sbuf, [ridx, cidx])
                xd = plsc.load_gather(dbuf, [ridx, cidx])
                comp.append(xd - xs)
            d2 = comp[0] * comp[0] + comp[1] * comp[1] + comp[2] * comp[2]
            r = _rsqrt_sc(d2)
            f = 1.0 / (d2 * r + 1e-6)
            for cc in range(3):
                planes[grow + cc, pl.ds(lo, 16)] = comp[cc] * f
            return carry2
        lax.fori_loop(0, KBA * 8, grp, 0)
        pltpu.sync_copy(planes, posU_hbm.at[pl.ds(3 * r0, 3 * KBA)])
        return carry

    lax.fori_loop(0, RPW // KBA, block, 0)


@jax.jit
def _gather_call(pos16, srcv, dstv):
    f = pl.kernel(
        _gather_body,
        out_type=jax.ShapeDtypeStruct((3 * NROWS, ROW), jnp.float32),
        mesh=plsc.VectorSubcoreMesh(core_axis_name="c", subcore_axis_name="s"),
        scratch_types=[
            pltpu.VMEM((KBA * ROW,), jnp.int32),
            pltpu.VMEM((KBA * ROW,), jnp.int32),
            pltpu.VMEM((KBA * ROW, PW), jnp.float32),
            pltpu.VMEM((KBA * ROW, PW), jnp.float32),
            pltpu.VMEM((3 * KBA, ROW), jnp.float32),
            pltpu.VMEM_SHARED((N, PW), jnp.float32),
            pltpu.SemaphoreType.DMA,
        ],
        compiler_params=_SC_PARAMS,
    )
    return f(pos16, srcv, dstv)


BE = 4096


def _mlp_body(w1_ref, b1_ref, gam_ref, bet_ref, w2_ref, b2_ref,
              ul_ref, ur_ref, o_ref):
    w1 = w1_ref[...]
    w2 = w2_ref[...]
    b1 = b1_ref[...]
    gam = gam_ref[...]
    bet = bet_ref[...]
    b2 = b2_ref[...]

    def half(u_ref):
        hs = []
        for g in range(BE // ROW):
            ug = u_ref[3 * g:3 * g + 3, :]
            hs.append(lax.dot_general(ug, w1, (((0,), (0,)), ((), ())),
                                      preferred_element_type=jnp.float32))
        h = jnp.concatenate(hs, axis=0) + b1
        mu = jnp.mean(h, axis=1, keepdims=True)
        cen = h - mu
        var = jnp.mean(cen * cen, axis=1, keepdims=True)
        hn = cen * lax.rsqrt(var + 1e-5) * gam + bet
        g = 0.5 * hn * (1.0 + lax.erf(hn * 0.7071067811865476))
        return jnp.dot(g, w2, preferred_element_type=jnp.float32) + b2

    o_ref[...] = jnp.concatenate([half(ul_ref), half(ur_ref)], axis=1)


@jax.jit
def _mlp_call(W1, b1, gamma, beta, W2, b2, posU):
    grid = (CROWS // BE,)
    full = lambda i: (0, 0)
    return pl.pallas_call(
        _mlp_body,
        grid=grid,
        in_specs=[
            pl.BlockSpec((3, H2), full),
            pl.BlockSpec((1, H2), full),
            pl.BlockSpec((1, H2), full),
            pl.BlockSpec((1, H2), full),
            pl.BlockSpec((H2, H2), full),
            pl.BlockSpec((1, H2), full),
            pl.BlockSpec((3 * BE // ROW, ROW), lambda i: (i, 0)),
            pl.BlockSpec((3 * BE // ROW, ROW),
                         lambda i: (i + CROWS // BE, 0)),
        ],
        out_specs=pl.BlockSpec((BE, 2 * H2), lambda i: (i, 0)),
        out_shape=jax.ShapeDtypeStruct((CROWS, 2 * H2), jnp.float32),
    )(W1, b1.reshape(1, H2), gamma.reshape(1, H2), beta.reshape(1, H2),
      W2, b2.reshape(1, H2), posU, posU)


def _scatter_body(codes_hbm, dst_hbm, out_hbm,
                  idx2, upd2, ones_v, obuf, degb, acc, deg, sem, sem2):
    c = lax.axis_index("c")
    t = lax.axis_index("s")
    base = c * HALF
    iota = lax.iota(jnp.int32, 16)
    zero16 = jnp.zeros((16,), jnp.float32)
    one16 = jnp.full((16,), 1.0, jnp.float32)

    def initrow(r, carry):
        for g in range(4):
            obuf[r, pl.ds(g * 16, 16)] = zero16
        return carry
    lax.fori_loop(0, CB, initrow, 0)
    for k in range(CB // 16):
        degb[pl.ds(k * 16, 16)] = zero16
    for k in range(ROW // 16):
        ones_v[pl.ds(k * 16, 16)] = one16

    def zchunk(i, carry):
        cs = jnp.minimum(t * ZSTRIDE + i * CB, ACC_ROWS - CB)
        pltpu.sync_copy(obuf, acc.at[pl.ds(cs, CB)])
        pltpu.sync_copy(degb, deg.at[pl.ds(cs, CB)])
        return carry
    lax.fori_loop(0, ZCH, zchunk, 0)
    plsc.subcore_barrier()

    def remap(idx):
        for i in range(8):
            v = idx[pl.ds(i * 16, 16)]
            inr = jnp.logical_and(v >= base, v < base + HALF)
            dumped = BIGROW + i * 16 + iota
            idx[pl.ds(i * 16, 16)] = jnp.where(inr, v - base, dumped)

    M = 2 * CRPT

    def eoff(m):
        jj = t * CRPT + (m >> 1)
        side = m & 1
        return jj * ROW + side * (EPAD // 2), side * H2, jj

    def isactive(m):
        jj = t * CRPT + (m >> 1)
        return jnp.logical_or((m & 1) == 0, jj < RREAL)

    def issue_loads(m, k):
        e0, c0, jj = eoff(m)
        pltpu.async_copy(dst_hbm.at[pl.ds(e0, ROW)], idx2.at[k], sem)
        pltpu.async_copy(
            codes_hbm.at[pl.ds(jj * ROW, ROW), pl.ds(c0, H2)],
            upd2.at[k], sem)

    def wait_loads(m, k):
        e0, c0, jj = eoff(m)
        pltpu.make_async_copy(
            dst_hbm.at[pl.ds(e0, ROW)], idx2.at[k], sem).wait()
        pltpu.make_async_copy(
            codes_hbm.at[pl.ds(jj * ROW, ROW), pl.ds(c0, H2)],
            upd2.at[k], sem).wait()

    def issue_scats(k):
        pltpu.async_copy(upd2.at[k], acc.at[idx2.at[k]], sem2, add=True)
        pltpu.async_copy(ones_v, deg.at[idx2.at[k]], sem2, add=True)

    def wait_scats(k):
        pltpu.make_async_copy(upd2.at[k], acc.at[idx2.at[k]], sem2).wait()
        pltpu.make_async_copy(ones_v, deg.at[idx2.at[k]], sem2).wait()

    @pl.when(isactive(0))
    def _():
        issue_loads(0, 0)

    @pl.when(isactive(1))
    def _():
        issue_loads(1, 1)

    def sloop(m, carry):
        k = lax.rem(m, 3)
        act = isactive(m)

        @pl.when(act)
        def _():
            wait_loads(m, k)
            remap(idx2.at[k])

        @pl.when(jnp.logical_and(m >= 1, isactive(m - 1)))
        def _():
            wait_scats(lax.rem(m + 2, 3))

        @pl.when(act)
        def _():
            issue_scats(k)

        @pl.when(jnp.logical_and(m + 2 < M, isactive(m + 2)))
        def _():
            issue_loads(m + 2, lax.rem(m + 2, 3))
        return carry

    lax.fori_loop(0, M, sloop, 0)

    @pl.when(isactive(M - 1))
    def _():
        wait_scats(lax.rem(M - 1, 3))
    plsc.subcore_barrier()

    def dchunk(i, carry):
        cs = jnp.minimum(t * TSTRIDE + i * CB, HALF - CB)
        pltpu.sync_copy(acc.at[pl.ds(cs, CB)], obuf)
        pltpu.sync_copy(deg.at[pl.ds(cs, CB)], degb)

        def drow(r, carry2):
            dvec = plsc.load_gather(degb, [jnp.full((16,), r, jnp.int32)])
            rec = 1.0 / jnp.maximum(dvec, 1.0)
            for g in range(4):
                obuf[r, pl.ds(g * 16, 16)] = obuf[r, pl.ds(g * 16, 16)] * rec
            return carry2
        lax.fori_loop(0, CB, drow, 0)
        pltpu.sync_copy(obuf, out_hbm.at[pl.ds(base + cs, CB)])
        return carry
    lax.fori_loop(0, DCH, dchunk, 0)


@jax.jit
def _scatter_call(codes, dstv):
    f = pl.kernel(
        _scatter_body,
        out_type=jax.ShapeDtypeStruct((N, H2), jnp.float32),
        mesh=plsc.VectorSubcoreMesh(core_axis_name="c", subcore_axis_name="s"),
        scratch_types=[
            pltpu.VMEM((3, ROW), jnp.int32),
            pltpu.VMEM((3, ROW, H2), jnp.float32),
            pltpu.VMEM((ROW,), jnp.float32),
            pltpu.VMEM((CB, H2), jnp.float32),
            pltpu.VMEM((CB,), jnp.float32),
            pltpu.VMEM_SHARED((ACC_ROWS, H2), jnp.float32),
            pltpu.VMEM_SHARED((ACC_ROWS,), jnp.float32),
            pltpu.SemaphoreType.DMA,
            pltpu.SemaphoreType.DMA,
        ],
        compiler_params=_SC_PARAMS,
    )
    return f(codes, dstv)


def kernel(pos, edge_index, batch, W1, b1, gamma, beta, W2, b2):
    srcv = edge_index[0]
    dstv = edge_index[1]
    pos16 = jnp.pad(pos, ((0, 0), (0, PW - 3)))
    posU = _gather_call(pos16, srcv, dstv)
    codes = _mlp_call(W1, b1, gamma, beta, W2, b2, posU)
    return _scatter_call(codes, dstv)
